# Initial kernel scaffold; baseline (speedup 1.0000x reference)
import functools

import jax
import jax.numpy as jnp
from jax import lax
from jax.experimental import pallas as pl
from jax.experimental.pallas import tpu as pltpu
from jax.experimental.pallas import tpu_sc as plsc

NC = 2
NS = 16
NW = NC * NS
L = 16

CH = 128
K = 80
EW = K * CH
E_PAD = NW * EW
NP = 10240
RPW = NP // NS
D = 16

_mesh = plsc.VectorSubcoreMesh(core_axis_name="c", subcore_axis_name="s")


def _deg_body(dst_hbm, out_hbm, dst_v, hist_v):
    c = lax.axis_index("c")
    s = lax.axis_index("s")
    wid = c * NS + s
    pltpu.sync_copy(dst_hbm.at[wid], dst_v)

    zero16 = jnp.zeros((L,), jnp.float32)
    ones16 = jnp.ones((L,), jnp.float32)

    def zero_step(i, carry):
        hist_v[pl.ds(i * L, L)] = zero16
        return carry

    lax.fori_loop(0, NP // L, zero_step, 0)

    def count_step(i, carry):
        idx = dst_v[pl.ds(i * L, L)]
        plsc.addupdate_scatter(hist_v, [idx], ones16)
        return carry

    lax.fori_loop(0, EW // L, count_step, 0)
    pltpu.sync_copy(hist_v, out_hbm.at[wid])


_deg_kernel = functools.partial(
    pl.kernel,
    out_type=jax.ShapeDtypeStruct((NW, NP), jnp.float32),
    mesh=_mesh,
    scratch_types=[
        pltpu.VMEM((EW,), jnp.int32),
        pltpu.VMEM((NP,), jnp.float32),
    ],
)(_deg_body)


def _agg_body(y_hbm, src_hbm, dst_hbm, out_hbm, src_v, dst_v, buf_v, stage_v,
              acc_s, gsem):
    c = lax.axis_index("c")
    s = lax.axis_index("s")
    wid = c * NS + s

    zero16 = jnp.zeros((L,), jnp.float32)

    def zero_step(i, carry):
        stage_v[i, :] = zero16
        return carry

    lax.fori_loop(0, RPW, zero_step, 0)
    pltpu.sync_copy(stage_v, acc_s.at[pl.ds(s * RPW, RPW)])

    pltpu.sync_copy(src_hbm.at[wid], src_v)
    pltpu.sync_copy(dst_hbm.at[wid], dst_v)
    plsc.subcore_barrier()

    def chunk_step(j, carry):
        pltpu.async_copy(y_hbm.at[src_v.at[j]], buf_v, gsem).wait()
        pltpu.sync_copy(buf_v, acc_s.at[dst_v.at[j]], add=True)
        return carry

    lax.fori_loop(0, K, chunk_step, 0)
    plsc.subcore_barrier()

    pltpu.sync_copy(acc_s.at[pl.ds(s * RPW, RPW)], stage_v)
    pltpu.sync_copy(stage_v, out_hbm.at[c, pl.ds(s * RPW, RPW)])


_agg_kernel = functools.partial(
    pl.kernel,
    out_type=jax.ShapeDtypeStruct((NC, NP, D), jnp.float32),
    mesh=_mesh,
    scratch_types=[
        pltpu.VMEM((K, CH), jnp.int32),
        pltpu.VMEM((K, CH), jnp.int32),
        pltpu.VMEM((CH, D), jnp.float32),
        pltpu.VMEM((RPW, D), jnp.float32),
        pltpu.VMEM_SHARED((NP, D), jnp.float32),
        pltpu.SemaphoreType.DMA,
    ],
)(_agg_body)


def _dinv_of(degp_block):
    return lax.rsqrt(jnp.sum(degp_block, axis=0) + 1.0)


def _y1_body(x_ref, w_ref, d_ref, y_ref):
    dinv = _dinv_of(d_ref[...])
    y_ref[...] = jnp.dot(
        x_ref[...], w_ref[...], preferred_element_type=jnp.float32
    ) * dinv[:, None]


def _mid_body(a0_ref, a1_ref, y_ref, d_ref, b1_ref, y2_ref):
    dinv = _dinv_of(d_ref[...])[:, None]
    out1 = dinv * (a0_ref[...] + a1_ref[...] + y_ref[...])
    h = jnp.maximum(out1 + b1_ref[...], 0.0)
    y2_ref[...] = dinv * h


def _out_body(a0_ref, a1_ref, y2_ref, d_ref, w2_ref, b2_ref, o_ref):
    dinv = _dinv_of(d_ref[...])[:, None]
    agg = dinv * (a0_ref[...] + a1_ref[...] + y2_ref[...])
    o_ref[...] = jnp.dot(
        agg, w2_ref[...], preferred_element_type=jnp.float32
    ) + b2_ref[...]


def kernel(x, edge_index, W1, b1, W2, b2):
    n, d_in = x.shape
    e = edge_index.shape[1]
    d_hid = W1.shape[1]
    d_out = W2.shape[1]

    src = edge_index[0].astype(jnp.int32)
    dst = edge_index[1].astype(jnp.int32)
    pad = jnp.full((E_PAD - e,), n, jnp.int32)
    srcp = jnp.concatenate([src, pad]).reshape(NW, K, CH)
    dstp = jnp.concatenate([dst, pad]).reshape(NW, K, CH)

    degp = _deg_kernel(dstp.reshape(NW, EW))[:, :n]

    blk = 1000
    grid = (n // blk,)

    y1 = pl.pallas_call(
        _y1_body,
        grid=grid,
        in_specs=[
            pl.BlockSpec((blk, d_in), lambda i: (i, 0)),
            pl.BlockSpec((d_in, d_hid), lambda i: (0, 0)),
            pl.BlockSpec((NW, blk), lambda i: (0, i)),
        ],
        out_specs=pl.BlockSpec((blk, d_hid), lambda i: (i, 0)),
        out_shape=jax.ShapeDtypeStruct((n, d_hid), jnp.float32),
    )(x, W1, degp)

    y1_pad = jnp.pad(y1, ((0, NP - n), (0, 0)))
    acc1 = _agg_kernel(y1_pad, srcp, dstp)

    y2 = pl.pallas_call(
        _mid_body,
        grid=grid,
        in_specs=[
            pl.BlockSpec((blk, d_hid), lambda i: (i, 0)),
            pl.BlockSpec((blk, d_hid), lambda i: (i, 0)),
            pl.BlockSpec((blk, d_hid), lambda i: (i, 0)),
            pl.BlockSpec((NW, blk), lambda i: (0, i)),
            pl.BlockSpec((1, d_hid), lambda i: (0, 0)),
        ],
        out_specs=pl.BlockSpec((blk, d_hid), lambda i: (i, 0)),
        out_shape=jax.ShapeDtypeStruct((n, d_hid), jnp.float32),
    )(acc1[0, :n], acc1[1, :n], y1, degp, b1.reshape(1, d_hid))

    y2_pad = jnp.pad(y2, ((0, NP - n), (0, 0)))
    acc2 = _agg_kernel(y2_pad, srcp, dstp)

    out = pl.pallas_call(
        _out_body,
        grid=grid,
        in_specs=[
            pl.BlockSpec((blk, d_hid), lambda i: (i, 0)),
            pl.BlockSpec((blk, d_hid), lambda i: (i, 0)),
            pl.BlockSpec((blk, d_hid), lambda i: (i, 0)),
            pl.BlockSpec((NW, blk), lambda i: (0, i)),
            pl.BlockSpec((d_hid, d_out), lambda i: (0, 0)),
            pl.BlockSpec((1, d_out), lambda i: (0, 0)),
        ],
        out_specs=pl.BlockSpec((blk, d_out), lambda i: (i, 0)),
        out_shape=jax.ShapeDtypeStruct((n, d_out), jnp.float32),
    )(acc2[0, :n], acc2[1, :n], y2, degp, W2, b2.reshape(1, d_out))

    return out


# speedup vs baseline: 27.9296x; 27.9296x over previous
import functools

import jax
import jax.numpy as jnp
from jax import lax
from jax.experimental import pallas as pl
from jax.experimental.pallas import tpu as pltpu
from jax.experimental.pallas import tpu_sc as plsc

NC = 2
NS = 16
NW = NC * NS
L = 16

CH = 128
K = 80
EW = K * CH
E_PAD = NW * EW
NP = 10240
RPW = NP // NS
D = 16

_mesh = plsc.VectorSubcoreMesh(core_axis_name="c", subcore_axis_name="s")
_sc_params = pltpu.CompilerParams(
    needs_layout_passes=False, use_tc_tiling_on_sc=False
)


def _deg_body(dst_hbm, out_hbm, dst_v, hist_v):
    c = lax.axis_index("c")
    s = lax.axis_index("s")
    wid = c * NS + s
    pltpu.sync_copy(dst_hbm.at[wid], dst_v)

    zero16 = jnp.zeros((L,), jnp.float32)
    ones16 = jnp.ones((L,), jnp.float32)

    def zero_step(i, carry):
        hist_v[pl.ds(i * L, L)] = zero16
        return carry

    lax.fori_loop(0, NP // L, zero_step, 0)

    def count_step(i, carry):
        idx = dst_v[pl.ds(i * L, L)]
        plsc.addupdate_scatter(hist_v, [idx], ones16)
        return carry

    lax.fori_loop(0, EW // L, count_step, 0)
    pltpu.sync_copy(hist_v, out_hbm.at[wid])


_deg_kernel = functools.partial(
    pl.kernel,
    out_type=jax.ShapeDtypeStruct((NW, NP), jnp.float32),
    mesh=_mesh,
    scratch_types=[
        pltpu.VMEM((EW,), jnp.int32),
        pltpu.VMEM((NP,), jnp.float32),
    ],
    compiler_params=_sc_params,
)(_deg_body)


def _agg_body(y_hbm, src_hbm, dst_hbm, out_hbm, src_v, dst_v, buf_v, stage_v,
              acc_s, gsem):
    c = lax.axis_index("c")
    s = lax.axis_index("s")
    wid = c * NS + s

    zero16 = jnp.zeros((L,), jnp.float32)

    def zero_step(i, carry):
        stage_v[i, :] = zero16
        return carry

    lax.fori_loop(0, RPW, zero_step, 0)
    pltpu.sync_copy(stage_v, acc_s.at[pl.ds(s * RPW, RPW)])

    pltpu.sync_copy(src_hbm.at[wid], src_v)
    pltpu.sync_copy(dst_hbm.at[wid], dst_v)
    plsc.subcore_barrier()

    def chunk_step(j, carry):
        pltpu.async_copy(y_hbm.at[src_v.at[j]], buf_v, gsem).wait()
        pltpu.sync_copy(buf_v, acc_s.at[dst_v.at[j]], add=True)
        return carry

    lax.fori_loop(0, K, chunk_step, 0)
    plsc.subcore_barrier()

    pltpu.sync_copy(acc_s.at[pl.ds(s * RPW, RPW)], stage_v)
    pltpu.sync_copy(stage_v, out_hbm.at[c, pl.ds(s * RPW, RPW)])


_agg_kernel = functools.partial(
    pl.kernel,
    out_type=jax.ShapeDtypeStruct((NC, NP, D), jnp.float32),
    mesh=_mesh,
    scratch_types=[
        pltpu.VMEM((K, CH), jnp.int32),
        pltpu.VMEM((K, CH), jnp.int32),
        pltpu.VMEM((CH, D), jnp.float32),
        pltpu.VMEM((RPW, D), jnp.float32),
        pltpu.VMEM_SHARED((NP, D), jnp.float32),
        pltpu.SemaphoreType.DMA,
    ],
    compiler_params=_sc_params,
)(_agg_body)


def _dinv_of(degp_block):
    return lax.rsqrt(jnp.sum(degp_block, axis=1) + 1.0)


def _y1_body(x_ref, w_ref, d_ref, y_ref):
    dinv = _dinv_of(d_ref[...])
    y_ref[...] = jnp.dot(
        x_ref[...], w_ref[...], preferred_element_type=jnp.float32
    ) * dinv[:, None]


_DEG_SPEC = pl.BlockSpec((1000, NW), lambda i: (i, 0))


def _mid_body(a0_ref, a1_ref, y_ref, d_ref, b1_ref, y2_ref):
    dinv = _dinv_of(d_ref[...])[:, None]
    out1 = dinv * (a0_ref[...] + a1_ref[...] + y_ref[...])
    h = jnp.maximum(out1 + b1_ref[...], 0.0)
    y2_ref[...] = dinv * h


def _out_body(a0_ref, a1_ref, y2_ref, d_ref, w2_ref, b2_ref, o_ref):
    dinv = _dinv_of(d_ref[...])[:, None]
    agg = dinv * (a0_ref[...] + a1_ref[...] + y2_ref[...])
    o_ref[...] = jnp.dot(
        agg, w2_ref[...], preferred_element_type=jnp.float32
    ) + b2_ref[...]


def kernel(x, edge_index, W1, b1, W2, b2):
    n, d_in = x.shape
    e = edge_index.shape[1]
    d_hid = W1.shape[1]
    d_out = W2.shape[1]

    src = edge_index[0].astype(jnp.int32)
    dst = edge_index[1].astype(jnp.int32)
    pad = jnp.full((E_PAD - e,), n, jnp.int32)
    srcp = jnp.concatenate([src, pad]).reshape(NW, K, CH)
    dstp = jnp.concatenate([dst, pad]).reshape(NW, K, CH)

    degp = _deg_kernel(dstp.reshape(NW, EW))[:, :n].T

    blk = 1000
    grid = (n // blk,)

    y1 = pl.pallas_call(
        _y1_body,
        grid=grid,
        in_specs=[
            pl.BlockSpec((blk, d_in), lambda i: (i, 0)),
            pl.BlockSpec((d_in, d_hid), lambda i: (0, 0)),
            pl.BlockSpec((blk, NW), lambda i: (i, 0)),
        ],
        out_specs=pl.BlockSpec((blk, d_hid), lambda i: (i, 0)),
        out_shape=jax.ShapeDtypeStruct((n, d_hid), jnp.float32),
    )(x, W1, degp)

    y1_pad = jnp.pad(y1, ((0, NP - n), (0, 0)))
    acc1 = _agg_kernel(y1_pad, srcp, dstp)

    y2 = pl.pallas_call(
        _mid_body,
        grid=grid,
        in_specs=[
            pl.BlockSpec((blk, d_hid), lambda i: (i, 0)),
            pl.BlockSpec((blk, d_hid), lambda i: (i, 0)),
            pl.BlockSpec((blk, d_hid), lambda i: (i, 0)),
            pl.BlockSpec((blk, NW), lambda i: (i, 0)),
            pl.BlockSpec((1, d_hid), lambda i: (0, 0)),
        ],
        out_specs=pl.BlockSpec((blk, d_hid), lambda i: (i, 0)),
        out_shape=jax.ShapeDtypeStruct((n, d_hid), jnp.float32),
    )(acc1[0, :n], acc1[1, :n], y1, degp, b1.reshape(1, d_hid))

    y2_pad = jnp.pad(y2, ((0, NP - n), (0, 0)))
    acc2 = _agg_kernel(y2_pad, srcp, dstp)

    out = pl.pallas_call(
        _out_body,
        grid=grid,
        in_specs=[
            pl.BlockSpec((blk, d_hid), lambda i: (i, 0)),
            pl.BlockSpec((blk, d_hid), lambda i: (i, 0)),
            pl.BlockSpec((blk, d_hid), lambda i: (i, 0)),
            pl.BlockSpec((blk, NW), lambda i: (i, 0)),
            pl.BlockSpec((d_hid, d_out), lambda i: (0, 0)),
            pl.BlockSpec((1, d_out), lambda i: (0, 0)),
        ],
        out_specs=pl.BlockSpec((blk, d_out), lambda i: (i, 0)),
        out_shape=jax.ShapeDtypeStruct((n, d_out), jnp.float32),
    )(acc2[0, :n], acc2[1, :n], y2, degp, W2, b2.reshape(1, d_out))

    return out


# speedup vs baseline: 39.2164x; 1.4041x over previous
import functools

import jax
import jax.numpy as jnp
from jax import lax
from jax.experimental import pallas as pl
from jax.experimental.pallas import tpu as pltpu
from jax.experimental.pallas import tpu_sc as plsc

NC = 2
NS = 16
NW = NC * NS
L = 16

CH = 128
K = 80
NBUF = 8
NGRP = K // NBUF
EW = K * CH
E_PAD = NW * EW
NP = 10240
RPW = NP // NS
D = 16

_mesh = plsc.VectorSubcoreMesh(core_axis_name="c", subcore_axis_name="s")
_sc_params = pltpu.CompilerParams(
    needs_layout_passes=False, use_tc_tiling_on_sc=False
)


def _deg_body(dst_hbm, out_hbm, dst_v, hist_v):
    c = lax.axis_index("c")
    s = lax.axis_index("s")
    wid = c * NS + s
    pltpu.sync_copy(dst_hbm.at[wid], dst_v)

    zero16 = jnp.zeros((L,), jnp.float32)
    ones16 = jnp.ones((L,), jnp.float32)

    def zero_step(i, carry):
        hist_v[pl.ds(i * L, L)] = zero16
        return carry

    lax.fori_loop(0, NP // L, zero_step, 0)

    def count_step(i, carry):
        idx = dst_v[pl.ds(i * L, L)]
        plsc.addupdate_scatter(hist_v, [idx], ones16)
        return carry

    lax.fori_loop(0, EW // L, count_step, 0)
    pltpu.sync_copy(hist_v, out_hbm.at[wid])


_deg_kernel = functools.partial(
    pl.kernel,
    out_type=jax.ShapeDtypeStruct((NW, NP), jnp.float32),
    mesh=_mesh,
    scratch_types=[
        pltpu.VMEM((EW,), jnp.int32),
        pltpu.VMEM((NP,), jnp.float32),
    ],
    compiler_params=_sc_params,
)(_deg_body)


def _agg_body(y_hbm, src_hbm, dst_hbm, out_hbm, src_v, dst_v, buf_v, stage_v,
              acc_s, gsem, ssem):
    c = lax.axis_index("c")
    s = lax.axis_index("s")
    wid = c * NS + s

    zero16 = jnp.zeros((L,), jnp.float32)

    def zero_step(i, carry):
        stage_v[i, :] = zero16
        return carry

    lax.fori_loop(0, RPW, zero_step, 0)
    pltpu.sync_copy(stage_v, acc_s.at[pl.ds(s * RPW, RPW)])

    pltpu.sync_copy(src_hbm.at[wid], src_v)
    pltpu.sync_copy(dst_hbm.at[wid], dst_v)
    plsc.subcore_barrier()

    def gather(j, b):
        return pltpu.async_copy(y_hbm.at[src_v.at[j]], buf_v.at[b],
                                gsem.at[b])

    def gather_wait(j, b):
        pltpu.make_async_copy(y_hbm.at[src_v.at[j]], buf_v.at[b],
                              gsem.at[b]).wait()

    def scatter(j, b):
        return pltpu.async_copy(buf_v.at[b], acc_s.at[dst_v.at[j]],
                                ssem.at[b], add=True)

    def scatter_wait(j, b):
        pltpu.make_async_copy(buf_v.at[b], acc_s.at[dst_v.at[j]],
                              ssem.at[b]).wait()

    for b in range(NBUF):
        gather(b, b)

    def grp_step(g, carry):
        base = g * NBUF
        for b in range(NBUF):
            gather_wait(base + b, b)
            scatter(base + b, b)
        for b in range(NBUF):
            scatter_wait(base + b, b)

            @pl.when(g < NGRP - 1)
            def _():
                gather(base + NBUF + b, b)

        return carry

    lax.fori_loop(0, NGRP, grp_step, 0)
    plsc.subcore_barrier()

    pltpu.sync_copy(acc_s.at[pl.ds(s * RPW, RPW)], stage_v)
    pltpu.sync_copy(stage_v, out_hbm.at[c, pl.ds(s * RPW, RPW)])


_agg_kernel = functools.partial(
    pl.kernel,
    out_type=jax.ShapeDtypeStruct((NC, NP, D), jnp.float32),
    mesh=_mesh,
    scratch_types=[
        pltpu.VMEM((K, CH), jnp.int32),
        pltpu.VMEM((K, CH), jnp.int32),
        pltpu.VMEM((NBUF, CH, D), jnp.float32),
        pltpu.VMEM((RPW, D), jnp.float32),
        pltpu.VMEM_SHARED((NP, D), jnp.float32),
        pltpu.SemaphoreType.DMA((NBUF,)),
        pltpu.SemaphoreType.DMA((NBUF,)),
    ],
    compiler_params=_sc_params,
)(_agg_body)


_BLK = 1000


def _dinv_blk(d_ref):
    return lax.rsqrt(jnp.sum(d_ref[...], axis=1) + 1.0)


def _y1_body(x_ref, w_ref, d_ref, y_ref):
    dinv = _dinv_blk(d_ref)
    y_ref[...] = jnp.dot(
        x_ref[...], w_ref[...], preferred_element_type=jnp.float32
    ) * dinv[:, None]


def _mid_body(a0_ref, a1_ref, y_ref, d_ref, b1_ref, y2_ref):
    dinv = _dinv_blk(d_ref)[:, None]
    out1 = dinv * (a0_ref[0] + a1_ref[0] + y_ref[...])
    h = jnp.maximum(out1 + b1_ref[...], 0.0)
    y2_ref[...] = dinv * h


def _out_body(a0_ref, a1_ref, y2_ref, d_ref, w2_ref, b2_ref, o_ref):
    dinv = _dinv_blk(d_ref)[:, None]
    agg = dinv * (a0_ref[0] + a1_ref[0] + y2_ref[...])
    o_ref[...] = jnp.dot(
        agg, w2_ref[...], preferred_element_type=jnp.float32
    ) + b2_ref[...]


def kernel(x, edge_index, W1, b1, W2, b2):
    n, d_in = x.shape
    e = edge_index.shape[1]
    d_hid = W1.shape[1]
    d_out = W2.shape[1]

    src = edge_index[0].astype(jnp.int32)
    dst = edge_index[1].astype(jnp.int32)
    pad = jnp.full((E_PAD - e,), n, jnp.int32)
    srcp = jnp.concatenate([src, pad]).reshape(NW, K, CH)
    dstp = jnp.concatenate([dst, pad]).reshape(NW, K, CH)

    degp = _deg_kernel(dstp.reshape(NW, EW)).T

    grid = (n // _BLK,)
    deg_spec = pl.BlockSpec((_BLK, NW), lambda i: (i, 0))
    hid_spec = pl.BlockSpec((_BLK, d_hid), lambda i: (i, 0))
    acc0_spec = pl.BlockSpec((1, _BLK, d_hid), lambda i: (0, i, 0))
    acc1_spec = pl.BlockSpec((1, _BLK, d_hid), lambda i: (1, i, 0))

    y1 = pl.pallas_call(
        _y1_body,
        grid=grid,
        in_specs=[
            pl.BlockSpec((_BLK, d_in), lambda i: (i, 0)),
            pl.BlockSpec((d_in, d_hid), lambda i: (0, 0)),
            deg_spec,
        ],
        out_specs=hid_spec,
        out_shape=jax.ShapeDtypeStruct((NP, d_hid), jnp.float32),
    )(x, W1, degp)

    acc1 = _agg_kernel(y1, srcp, dstp)

    y2 = pl.pallas_call(
        _mid_body,
        grid=grid,
        in_specs=[
            acc0_spec,
            acc1_spec,
            hid_spec,
            deg_spec,
            pl.BlockSpec((1, d_hid), lambda i: (0, 0)),
        ],
        out_specs=hid_spec,
        out_shape=jax.ShapeDtypeStruct((NP, d_hid), jnp.float32),
    )(acc1, acc1, y1, degp, b1.reshape(1, d_hid))

    acc2 = _agg_kernel(y2, srcp, dstp)

    out = pl.pallas_call(
        _out_body,
        grid=grid,
        in_specs=[
            acc0_spec,
            acc1_spec,
            hid_spec,
            deg_spec,
            pl.BlockSpec((d_hid, d_out), lambda i: (0, 0)),
            pl.BlockSpec((1, d_out), lambda i: (0, 0)),
        ],
        out_specs=pl.BlockSpec((_BLK, d_out), lambda i: (i, 0)),
        out_shape=jax.ShapeDtypeStruct((n, d_out), jnp.float32),
    )(acc2, acc2, y2, degp, W2, b2.reshape(1, d_out))

    return out


# speedup vs baseline: 55.1740x; 1.4069x over previous
import functools

import jax
import jax.numpy as jnp
from jax import lax
from jax.experimental import pallas as pl
from jax.experimental.pallas import tpu as pltpu
from jax.experimental.pallas import tpu_sc as plsc

NC = 2
NS = 16
NW = NC * NS
L = 16

CH = 128
K = 80
NBUF = 8
NGRP = K // NBUF
EW = K * CH
E_PAD = NW * EW
NP = 10240
RPW = NP // NS
D = 16

_mesh = plsc.VectorSubcoreMesh(core_axis_name="c", subcore_axis_name="s")
_sc_params = pltpu.CompilerParams(
    needs_layout_passes=False, use_tc_tiling_on_sc=False
)


def _deg_body(dst_hbm, out_hbm, dst_v, hist_v):
    c = lax.axis_index("c")
    s = lax.axis_index("s")
    wid = c * NS + s
    pltpu.sync_copy(dst_hbm.at[wid], dst_v)

    zero16 = jnp.zeros((L,), jnp.float32)
    ones16 = jnp.ones((L,), jnp.float32)

    def zero_step(i, carry):
        hist_v[pl.ds(i * L, L)] = zero16
        return carry

    lax.fori_loop(0, NP // L, zero_step, 0)

    def count_step(i, carry):
        idx = dst_v[pl.ds(i * L, L)]
        plsc.addupdate_scatter(hist_v, [idx], ones16)
        return carry

    lax.fori_loop(0, EW // L, count_step, 0)
    pltpu.sync_copy(hist_v, out_hbm.at[wid])


_deg_kernel = functools.partial(
    pl.kernel,
    out_type=jax.ShapeDtypeStruct((NW, NP), jnp.float32),
    mesh=_mesh,
    scratch_types=[
        pltpu.VMEM((EW,), jnp.int32),
        pltpu.VMEM((NP,), jnp.float32),
    ],
    compiler_params=_sc_params,
)(_deg_body)


def _agg_body(y_hbm, src_hbm, dst_hbm, out_hbm, src_v, dst_v, buf_v, stage_v,
              y_s, acc_s, gsem, ssem):
    c = lax.axis_index("c")
    s = lax.axis_index("s")
    wid = c * NS + s

    pltpu.sync_copy(y_hbm.at[pl.ds(s * RPW, RPW)], y_s.at[pl.ds(s * RPW, RPW)])

    zero16 = jnp.zeros((L,), jnp.float32)

    def zero_step(i, carry):
        stage_v[i, :] = zero16
        return carry

    lax.fori_loop(0, RPW, zero_step, 0)
    pltpu.sync_copy(stage_v, acc_s.at[pl.ds(s * RPW, RPW)])

    pltpu.sync_copy(src_hbm.at[wid], src_v)
    pltpu.sync_copy(dst_hbm.at[wid], dst_v)
    plsc.subcore_barrier()

    def gather(j, b):
        return pltpu.async_copy(y_s.at[src_v.at[j]], buf_v.at[b],
                                gsem.at[b])

    def gather_wait(j, b):
        pltpu.make_async_copy(y_s.at[src_v.at[j]], buf_v.at[b],
                              gsem.at[b]).wait()

    def scatter(j, b):
        return pltpu.async_copy(buf_v.at[b], acc_s.at[dst_v.at[j]],
                                ssem.at[b], add=True)

    def scatter_wait(j, b):
        pltpu.make_async_copy(buf_v.at[b], acc_s.at[dst_v.at[j]],
                              ssem.at[b]).wait()

    for b in range(NBUF):
        gather(b, b)

    def grp_step(g, carry):
        base = g * NBUF
        for b in range(NBUF):
            gather_wait(base + b, b)
            scatter(base + b, b)
        for b in range(NBUF):
            scatter_wait(base + b, b)

            @pl.when(g < NGRP - 1)
            def _():
                gather(base + NBUF + b, b)

        return carry

    lax.fori_loop(0, NGRP, grp_step, 0)
    plsc.subcore_barrier()

    pltpu.sync_copy(acc_s.at[pl.ds(s * RPW, RPW)], stage_v)
    pltpu.sync_copy(stage_v, out_hbm.at[c, pl.ds(s * RPW, RPW)])


_agg_kernel = functools.partial(
    pl.kernel,
    out_type=jax.ShapeDtypeStruct((NC, NP, D), jnp.float32),
    mesh=_mesh,
    scratch_types=[
        pltpu.VMEM((K, CH), jnp.int32),
        pltpu.VMEM((K, CH), jnp.int32),
        pltpu.VMEM((NBUF, CH, D), jnp.float32),
        pltpu.VMEM((RPW, D), jnp.float32),
        pltpu.VMEM_SHARED((NP, D), jnp.float32),
        pltpu.VMEM_SHARED((NP, D), jnp.float32),
        pltpu.SemaphoreType.DMA((NBUF,)),
        pltpu.SemaphoreType.DMA((NBUF,)),
    ],
    compiler_params=_sc_params,
)(_agg_body)


_BLK = 1000


def _dinv_blk(d_ref):
    return lax.rsqrt(jnp.sum(d_ref[...], axis=1) + 1.0)


def _y1_body(x_ref, w_ref, d_ref, y_ref):
    dinv = _dinv_blk(d_ref)
    y_ref[...] = jnp.dot(
        x_ref[...], w_ref[...], preferred_element_type=jnp.float32
    ) * dinv[:, None]


def _mid_body(a0_ref, a1_ref, y_ref, d_ref, b1_ref, y2_ref):
    dinv = _dinv_blk(d_ref)[:, None]
    out1 = dinv * (a0_ref[0] + a1_ref[0] + y_ref[...])
    h = jnp.maximum(out1 + b1_ref[...], 0.0)
    y2_ref[...] = dinv * h


def _out_body(a0_ref, a1_ref, y2_ref, d_ref, w2_ref, b2_ref, o_ref):
    dinv = _dinv_blk(d_ref)[:, None]
    agg = dinv * (a0_ref[0] + a1_ref[0] + y2_ref[...])
    o_ref[...] = jnp.dot(
        agg, w2_ref[...], preferred_element_type=jnp.float32
    ) + b2_ref[...]


def kernel(x, edge_index, W1, b1, W2, b2):
    n, d_in = x.shape
    e = edge_index.shape[1]
    d_hid = W1.shape[1]
    d_out = W2.shape[1]

    src = edge_index[0].astype(jnp.int32)
    dst = edge_index[1].astype(jnp.int32)
    pad = jnp.full((E_PAD - e,), n, jnp.int32)
    srcp = jnp.concatenate([src, pad]).reshape(NW, K, CH)
    dstp = jnp.concatenate([dst, pad]).reshape(NW, K, CH)

    degp = _deg_kernel(dstp.reshape(NW, EW)).T

    grid = (n // _BLK,)
    deg_spec = pl.BlockSpec((_BLK, NW), lambda i: (i, 0))
    hid_spec = pl.BlockSpec((_BLK, d_hid), lambda i: (i, 0))
    acc0_spec = pl.BlockSpec((1, _BLK, d_hid), lambda i: (0, i, 0))
    acc1_spec = pl.BlockSpec((1, _BLK, d_hid), lambda i: (1, i, 0))

    y1 = pl.pallas_call(
        _y1_body,
        grid=grid,
        in_specs=[
            pl.BlockSpec((_BLK, d_in), lambda i: (i, 0)),
            pl.BlockSpec((d_in, d_hid), lambda i: (0, 0)),
            deg_spec,
        ],
        out_specs=hid_spec,
        out_shape=jax.ShapeDtypeStruct((NP, d_hid), jnp.float32),
    )(x, W1, degp)

    acc1 = _agg_kernel(y1, srcp, dstp)

    y2 = pl.pallas_call(
        _mid_body,
        grid=grid,
        in_specs=[
            acc0_spec,
            acc1_spec,
            hid_spec,
            deg_spec,
            pl.BlockSpec((1, d_hid), lambda i: (0, 0)),
        ],
        out_specs=hid_spec,
        out_shape=jax.ShapeDtypeStruct((NP, d_hid), jnp.float32),
    )(acc1, acc1, y1, degp, b1.reshape(1, d_hid))

    acc2 = _agg_kernel(y2, srcp, dstp)

    out = pl.pallas_call(
        _out_body,
        grid=grid,
        in_specs=[
            acc0_spec,
            acc1_spec,
            hid_spec,
            deg_spec,
            pl.BlockSpec((d_hid, d_out), lambda i: (0, 0)),
            pl.BlockSpec((1, d_out), lambda i: (0, 0)),
        ],
        out_specs=pl.BlockSpec((_BLK, d_out), lambda i: (i, 0)),
        out_shape=jax.ShapeDtypeStruct((n, d_out), jnp.float32),
    )(acc2, acc2, y2, degp, W2, b2.reshape(1, d_out))

    return out


# speedup vs baseline: 59.9455x; 1.0865x over previous
import functools

import jax
import jax.numpy as jnp
from jax import lax
from jax.experimental import pallas as pl
from jax.experimental.pallas import tpu as pltpu
from jax.experimental.pallas import tpu_sc as plsc

NC = 2
NS = 16
NW = NC * NS
L = 16

CH = 128
K = 80
NBUF = 8
NGRP = K // NBUF
EW = K * CH
E_PAD = NW * EW
NP = 10240
RPW = NP // NS
D = 16

_mesh = plsc.VectorSubcoreMesh(core_axis_name="c", subcore_axis_name="s")
_sc_params = pltpu.CompilerParams(
    needs_layout_passes=False, use_tc_tiling_on_sc=False
)


def _deg_body(dst_hbm, out_hbm, dst_v, hist_v):
    c = lax.axis_index("c")
    s = lax.axis_index("s")
    wid = c * NS + s
    pltpu.sync_copy(dst_hbm.at[wid], dst_v)

    zero16 = jnp.zeros((L,), jnp.float32)
    ones16 = jnp.ones((L,), jnp.float32)

    def zero_step(i, carry):
        hist_v[pl.ds(i * L, L)] = zero16
        return carry

    lax.fori_loop(0, NP // L, zero_step, 0)

    def count_step(i, carry):
        idx = dst_v[pl.ds(i * L, L)]
        plsc.addupdate_scatter(hist_v, [idx], ones16)
        return carry

    lax.fori_loop(0, EW // L, count_step, 0)
    pltpu.sync_copy(hist_v, out_hbm.at[wid])


_deg_kernel = functools.partial(
    pl.kernel,
    out_type=jax.ShapeDtypeStruct((NW, NP), jnp.float32),
    mesh=_mesh,
    scratch_types=[
        pltpu.VMEM((EW,), jnp.int32),
        pltpu.VMEM((NP,), jnp.float32),
    ],
    compiler_params=_sc_params,
)(_deg_body)


def _agg_body(y_hbm, src_hbm, dst_hbm, out_hbm, src_v, dst_v, buf_v, stage_v,
              y_s, acc_s, gsem, ssem):
    c = lax.axis_index("c")
    s = lax.axis_index("s")
    wid = c * NS + s

    pltpu.sync_copy(y_hbm.at[pl.ds(s * RPW, RPW)], y_s.at[pl.ds(s * RPW, RPW)])

    zero16 = jnp.zeros((L,), jnp.float32)

    def zero_step(i, carry):
        stage_v[i, :] = zero16
        return carry

    lax.fori_loop(0, RPW, zero_step, 0)
    pltpu.sync_copy(stage_v, acc_s.at[pl.ds(s * RPW, RPW)])

    pltpu.sync_copy(src_hbm.at[wid], src_v)
    pltpu.sync_copy(dst_hbm.at[wid], dst_v)
    plsc.subcore_barrier()

    def gather(j, b):
        return pltpu.async_copy(y_s.at[src_v.at[j]], buf_v.at[b],
                                gsem.at[b])

    def gather_wait(j, b):
        pltpu.make_async_copy(y_s.at[src_v.at[j]], buf_v.at[b],
                              gsem.at[b]).wait()

    def scatter(j, b):
        return pltpu.async_copy(buf_v.at[b], acc_s.at[dst_v.at[j]],
                                ssem.at[b], add=True)

    def scatter_wait(j, b):
        pltpu.make_async_copy(buf_v.at[b], acc_s.at[dst_v.at[j]],
                              ssem.at[b]).wait()

    for b in range(NBUF):
        gather(b, b)

    def grp_step(g, carry):
        base = g * NBUF
        for b in range(NBUF):
            gather_wait(base + b, b)
            scatter(base + b, b)
        for b in range(NBUF):
            scatter_wait(base + b, b)

            @pl.when(g < NGRP - 1)
            def _():
                gather(base + NBUF + b, b)

        return carry

    lax.fori_loop(0, NGRP, grp_step, 0)
    plsc.subcore_barrier()

    pltpu.sync_copy(acc_s.at[pl.ds(s * RPW, RPW)], stage_v)
    pltpu.sync_copy(stage_v, out_hbm.at[c, pl.ds(s * RPW, RPW)])


_agg_kernel = functools.partial(
    pl.kernel,
    out_type=jax.ShapeDtypeStruct((NC, NP, D), jnp.float32),
    mesh=_mesh,
    scratch_types=[
        pltpu.VMEM((K, CH), jnp.int32),
        pltpu.VMEM((K, CH), jnp.int32),
        pltpu.VMEM((NBUF, CH, D), jnp.float32),
        pltpu.VMEM((RPW, D), jnp.float32),
        pltpu.VMEM_SHARED((NP, D), jnp.float32),
        pltpu.VMEM_SHARED((NP, D), jnp.float32),
        pltpu.SemaphoreType.DMA((NBUF,)),
        pltpu.SemaphoreType.DMA((NBUF,)),
    ],
    compiler_params=_sc_params,
)(_agg_body)


_BLK = 1000


def _dinv_blk(d_ref):
    return lax.rsqrt(jnp.sum(d_ref[...], axis=1) + 1.0)


def _y1_body(x_ref, w_ref, d_ref, y_ref):
    dinv = _dinv_blk(d_ref)
    y_ref[...] = jnp.dot(
        x_ref[...], w_ref[...], preferred_element_type=jnp.float32
    ) * dinv[:, None]


def _mid_body(a0_ref, a1_ref, y_ref, d_ref, b1_ref, y2_ref):
    dinv = _dinv_blk(d_ref)[:, None]
    out1 = dinv * (a0_ref[0] + a1_ref[0] + y_ref[...])
    h = jnp.maximum(out1 + b1_ref[...], 0.0)
    y2_ref[...] = dinv * h


def _out_body(a0_ref, a1_ref, y2_ref, d_ref, w2_ref, b2_ref, o_ref):
    dinv = _dinv_blk(d_ref)[:, None]
    agg = dinv * (a0_ref[0] + a1_ref[0] + y2_ref[...])
    o_ref[...] = jnp.dot(
        agg, w2_ref[...], preferred_element_type=jnp.float32
    ) + b2_ref[...]


def kernel(x, edge_index, W1, b1, W2, b2):
    n, d_in = x.shape
    e = edge_index.shape[1]
    d_hid = W1.shape[1]
    d_out = W2.shape[1]

    src = edge_index[0].astype(jnp.int32)
    dst = edge_index[1].astype(jnp.int32)
    pad = n + jnp.arange(E_PAD - e, dtype=jnp.int32) % (NP - n)
    srcp = jnp.concatenate([src, pad]).reshape(NW, K, CH)
    dstp = jnp.concatenate([dst, pad]).reshape(NW, K, CH)

    degp = _deg_kernel(dstp.reshape(NW, EW)).T

    grid = (n // _BLK,)
    deg_spec = pl.BlockSpec((_BLK, NW), lambda i: (i, 0))
    hid_spec = pl.BlockSpec((_BLK, d_hid), lambda i: (i, 0))
    acc0_spec = pl.BlockSpec((1, _BLK, d_hid), lambda i: (0, i, 0))
    acc1_spec = pl.BlockSpec((1, _BLK, d_hid), lambda i: (1, i, 0))

    y1 = pl.pallas_call(
        _y1_body,
        grid=grid,
        in_specs=[
            pl.BlockSpec((_BLK, d_in), lambda i: (i, 0)),
            pl.BlockSpec((d_in, d_hid), lambda i: (0, 0)),
            deg_spec,
        ],
        out_specs=hid_spec,
        out_shape=jax.ShapeDtypeStruct((NP, d_hid), jnp.float32),
    )(x, W1, degp)

    acc1 = _agg_kernel(y1, srcp, dstp)

    y2 = pl.pallas_call(
        _mid_body,
        grid=grid,
        in_specs=[
            acc0_spec,
            acc1_spec,
            hid_spec,
            deg_spec,
            pl.BlockSpec((1, d_hid), lambda i: (0, 0)),
        ],
        out_specs=hid_spec,
        out_shape=jax.ShapeDtypeStruct((NP, d_hid), jnp.float32),
    )(acc1, acc1, y1, degp, b1.reshape(1, d_hid))

    acc2 = _agg_kernel(y2, srcp, dstp)

    out = pl.pallas_call(
        _out_body,
        grid=grid,
        in_specs=[
            acc0_spec,
            acc1_spec,
            hid_spec,
            deg_spec,
            pl.BlockSpec((d_hid, d_out), lambda i: (0, 0)),
            pl.BlockSpec((1, d_out), lambda i: (0, 0)),
        ],
        out_specs=pl.BlockSpec((_BLK, d_out), lambda i: (i, 0)),
        out_shape=jax.ShapeDtypeStruct((n, d_out), jnp.float32),
    )(acc2, acc2, y2, degp, W2, b2.reshape(1, d_out))

    return out
